# baseline (speedup 1.0000x reference)
import functools

import jax
import jax.numpy as jnp
from jax import lax
from jax.experimental import pallas as pl
from jax.experimental.pallas import tpu as pltpu
from jax.experimental.pallas import tpu_sc as plsc

NC = 2
NS = 16
NW = NC * NS
EB = 128
RB = 4


def _sc_mesh():
    return plsc.VectorSubcoreMesh(
        core_axis_name="c", subcore_axis_name="s", num_cores=NC, num_subcores=NS
    )


def _degree_body(dst_hbm, ones_hbm, zeros_hbm, out_hbm, idx_v, ones_v, acc_sh, sem):
    c = lax.axis_index("c")
    s = lax.axis_index("s")
    wid = c * NS + s
    k = idx_v.shape[0]
    stripe = acc_sh.shape[0] // NS

    pltpu.sync_copy(dst_hbm.at[wid], idx_v)
    pltpu.sync_copy(ones_hbm, ones_v)
    pltpu.sync_copy(zeros_hbm, acc_sh.at[pl.ds(s * stripe, stripe)])
    plsc.subcore_barrier()

    def fire(j, carry):
        pltpu.async_copy(ones_v, acc_sh.at[idx_v.at[j]], sem, add=True)
        return carry

    lax.fori_loop(0, k, fire, 0)

    def drain(j, carry):
        pltpu.make_async_copy(ones_v, acc_sh.at[idx_v.at[j]], sem).wait()
        return carry

    lax.fori_loop(0, k, drain, 0)
    plsc.subcore_barrier()
    pltpu.sync_copy(
        acc_sh.at[pl.ds(s * stripe, stripe)],
        out_hbm.at[c, pl.ds(s * stripe, stripe)],
    )


def _make_degree(n_pad, k):
    return pl.kernel(
        _degree_body,
        out_type=jax.ShapeDtypeStruct((NC, n_pad, 1), jnp.float32),
        mesh=_sc_mesh(),
        scratch_types=[
            pltpu.VMEM((k, EB), jnp.int32),
            pltpu.VMEM((EB, 1), jnp.float32),
            pltpu.VMEM_SHARED((n_pad, 1), jnp.float32),
            pltpu.SemaphoreType.DMA,
        ],
    )


def _scatter_body(table_hbm, src_hbm, dst_hbm, zeros_hbm, out_hbm,
                  sidx_v, didx_v, rows_a, rows_b, rows_c, rows_d, acc_sh,
                  gsem_a, gsem_b, gsem_c, gsem_d,
                  ssem_a, ssem_b, ssem_c, ssem_d):
    c = lax.axis_index("c")
    s = lax.axis_index("s")
    wid = c * NS + s
    k = sidx_v.shape[0]
    stripe = acc_sh.shape[0] // NS
    rows = [rows_a, rows_b, rows_c, rows_d]
    gsems = [gsem_a, gsem_b, gsem_c, gsem_d]
    ssems = [ssem_a, ssem_b, ssem_c, ssem_d]

    pltpu.sync_copy(src_hbm.at[wid], sidx_v)
    pltpu.sync_copy(dst_hbm.at[wid], didx_v)
    for r in range(RB):
        pltpu.async_copy(table_hbm.at[sidx_v.at[r]], rows[r], gsems[r])
    pltpu.sync_copy(zeros_hbm, acc_sh.at[pl.ds(s * stripe, stripe)])
    plsc.subcore_barrier()

    def body(g, carry):
        j0 = g * RB
        for r in range(RB):
            pltpu.make_async_copy(
                table_hbm.at[sidx_v.at[j0 + r]], rows[r], gsems[r]).wait()
            pltpu.async_copy(rows[r], acc_sh.at[didx_v.at[j0 + r]], ssems[r],
                             add=True)
        for r in range(RB):
            pltpu.make_async_copy(
                rows[r], acc_sh.at[didx_v.at[j0 + r]], ssems[r]).wait()

            @pl.when(j0 + RB + r < k)
            def _():
                pltpu.async_copy(
                    table_hbm.at[sidx_v.at[j0 + RB + r]], rows[r], gsems[r])

        return carry

    lax.fori_loop(0, k // RB, body, 0)
    plsc.subcore_barrier()
    pltpu.sync_copy(
        acc_sh.at[pl.ds(s * stripe, stripe)],
        out_hbm.at[c, pl.ds(s * stripe, stripe)],
    )


def _make_scatter(n_pad, feat, k):
    return pl.kernel(
        _scatter_body,
        out_type=jax.ShapeDtypeStruct((NC, n_pad, feat), jnp.float32),
        mesh=_sc_mesh(),
        scratch_types=(
            [
                pltpu.VMEM((k, EB), jnp.int32),
                pltpu.VMEM((k, EB), jnp.int32),
            ]
            + [pltpu.VMEM((EB, feat), jnp.float32)] * RB
            + [pltpu.VMEM_SHARED((n_pad, feat), jnp.float32)]
            + [pltpu.SemaphoreType.DMA] * (2 * RB)
        ),
        compiler_params=pltpu.CompilerParams(use_tc_tiling_on_sc=False),
    )


def _scatter_fs_body(table_hbm, src_hbm, dst_hbm, zeros_hbm, out_hbm,
                     sidx_v, didx_v, rows_a, rows_b, rows_c, rows_d, acc_sh,
                     gsem_a, gsem_b, gsem_c, gsem_d,
                     ssem_a, ssem_b, ssem_c, ssem_d):
    c = lax.axis_index("c")
    s = lax.axis_index("s")
    k = sidx_v.shape[0]
    n_pad, fh = acc_sh.shape
    stripe = n_pad // NS
    rows = [rows_a, rows_b, rows_c, rows_d]
    gsems = [gsem_a, gsem_b, gsem_c, gsem_d]
    ssems = [ssem_a, ssem_b, ssem_c, ssem_d]

    pltpu.sync_copy(src_hbm.at[c, s], sidx_v)
    pltpu.sync_copy(dst_hbm.at[s], didx_v)
    for r in range(RB):
        pltpu.async_copy(table_hbm.at[sidx_v.at[r]], rows[r], gsems[r])
    pltpu.sync_copy(zeros_hbm, acc_sh.at[pl.ds(s * stripe, stripe)])
    plsc.subcore_barrier()

    def body(g, carry):
        j0 = g * RB
        for r in range(RB):
            pltpu.make_async_copy(
                table_hbm.at[sidx_v.at[j0 + r]], rows[r], gsems[r]).wait()
            pltpu.async_copy(rows[r], acc_sh.at[didx_v.at[j0 + r]], ssems[r],
                             add=True)
        for r in range(RB):
            pltpu.make_async_copy(
                rows[r], acc_sh.at[didx_v.at[j0 + r]], ssems[r]).wait()

            @pl.when(j0 + RB + r < k)
            def _():
                pltpu.async_copy(
                    table_hbm.at[sidx_v.at[j0 + RB + r]], rows[r], gsems[r])

        return carry

    lax.fori_loop(0, k // RB, body, 0)
    plsc.subcore_barrier()
    pltpu.sync_copy(
        acc_sh.at[pl.ds(s * stripe, stripe)],
        out_hbm.at[pl.ds(s * stripe, stripe), pl.ds(c * fh, fh)],
    )


def _make_scatter_fs(n_pad, feat, k):
    fh = feat // NC
    return pl.kernel(
        _scatter_fs_body,
        out_type=jax.ShapeDtypeStruct((n_pad, feat), jnp.float32),
        mesh=_sc_mesh(),
        scratch_types=(
            [
                pltpu.VMEM((k, EB), jnp.int32),
                pltpu.VMEM((k, EB), jnp.int32),
            ]
            + [pltpu.VMEM((EB, fh), jnp.float32)] * RB
            + [pltpu.VMEM_SHARED((n_pad, fh), jnp.float32)]
            + [pltpu.SemaphoreType.DMA] * (2 * RB)
        ),
        compiler_params=pltpu.CompilerParams(use_tc_tiling_on_sc=False),
    )


def _prologue_body(deg_ref, x_ref, w_ref, dis_ref, t_ref):
    deg = deg_ref[0] + deg_ref[1] + 1.0
    dis = lax.rsqrt(deg)
    dis_ref[...] = dis
    hh = jnp.dot(x_ref[...], w_ref[0], preferred_element_type=jnp.float32) * dis
    t_ref[...] = hh[None]


def _mid_body(s_ref, t_ref, dis_ref, b_ref, w_ref, o_ref, *, blk, n_valid):
    i = pl.program_id(0)
    dis = dis_ref[...]
    h = jnp.concatenate([t_ref[0], t_ref[1]], axis=1)
    z = dis * (s_ref[...] + h) + b_ref[...]
    z = jnp.maximum(z, 0.0)
    rows = i * blk + lax.broadcasted_iota(jnp.int32, (blk, 1), 0)
    z = jnp.where(rows < n_valid, z, 0.0)
    o_ref[...] = jnp.dot(
        z, w_ref[...], preferred_element_type=jnp.float32
    ) * dis


def _epilogue_body(s_ref, h_ref, dis_ref, b_ref, o_ref, *, ncls):
    full = dis_ref[...] * (s_ref[0] + s_ref[1] + h_ref[...])
    o_ref[...] = full[:, :ncls] + b_ref[...]


def _row_spec(blk, width):
    return pl.BlockSpec((blk, width), lambda i: (i, 0))


def _const_spec(shape):
    return pl.BlockSpec(shape, lambda i: (0, 0))


def _run_prologue(degp, x, ws, blk, n_pad):
    fin = x.shape[1]
    fh = ws.shape[2]
    grid = n_pad // blk
    return pl.pallas_call(
        _prologue_body,
        grid=(grid, NC),
        in_specs=[
            pl.BlockSpec((NC, blk, 1), lambda i, c: (0, i, 0)),
            pl.BlockSpec((blk, fin), lambda i, c: (i, 0)),
            pl.BlockSpec((1, fin, fh), lambda i, c: (c, 0, 0)),
        ],
        out_specs=[
            pl.BlockSpec((blk, 1), lambda i, c: (i, 0)),
            pl.BlockSpec((1, blk, fh), lambda i, c: (c, i, 0)),
        ],
        out_shape=[
            jax.ShapeDtypeStruct((n_pad, 1), jnp.float32),
            jax.ShapeDtypeStruct((NC, n_pad, fh), jnp.float32),
        ],
    )(degp, x, ws)


def _run_mid(s1, table1, dis, b1, w2p, blk, n_valid):
    n_pad, h = s1.shape
    fh = table1.shape[2]
    cp = w2p.shape[1]
    grid = n_pad // blk
    return pl.pallas_call(
        functools.partial(_mid_body, blk=blk, n_valid=n_valid),
        grid=(grid,),
        in_specs=[
            _row_spec(blk, h),
            pl.BlockSpec((NC, blk, fh), lambda i: (0, i, 0)),
            _row_spec(blk, 1),
            _const_spec((1, h)),
            _const_spec((h, cp)),
        ],
        out_specs=_row_spec(blk, cp),
        out_shape=jax.ShapeDtypeStruct((n_pad, cp), jnp.float32),
    )(s1, table1, dis, b1, w2p)


def _run_epilogue(s_parts, h2p, dis, b2, blk, n):
    n_pad, cp = h2p.shape
    ncls = b2.shape[1]
    grid = n_pad // blk
    return pl.pallas_call(
        functools.partial(_epilogue_body, ncls=ncls),
        grid=(grid,),
        in_specs=[
            pl.BlockSpec((NC, blk, cp), lambda i: (0, i, 0)),
            _row_spec(blk, cp),
            _row_spec(blk, 1),
            _const_spec((1, ncls)),
        ],
        out_specs=pl.BlockSpec((blk, ncls), lambda i: (i, 0)),
        out_shape=jax.ShapeDtypeStruct((n, ncls), jnp.float32),
    )(s_parts, h2p, dis, b2)


def _round_up(a, b):
    return (a + b - 1) // b * b


def kernel(x, edge_index, W1, b1, W2, b2):
    n, fin = x.shape
    hid = W1.shape[1]
    ncls = W2.shape[1]
    e = edge_index.shape[1]

    blk = 1024
    n_pad = _round_up(n + 1, max(blk, NS * 8))
    e_pad = _round_up(e, NW * EB * RB)
    k = e_pad // (NW * EB)
    k2 = e_pad // (NS * EB)
    cp = _round_up(ncls, 16)
    fh = hid // NC

    src = edge_index[0].astype(jnp.int32)
    dst = edge_index[1].astype(jnp.int32)
    pad_idx = n + jnp.arange(e_pad - e, dtype=jnp.int32) % (n_pad - n)
    src_p = jnp.concatenate([src, pad_idx])
    dst_p = jnp.concatenate([dst, pad_idx])
    src3 = src_p.reshape(NW, k, EB)
    dst3 = dst_p.reshape(NW, k, EB)
    src_fs = jnp.stack([src_p, src_p + n_pad]).reshape(NC, NS, k2, EB)
    dst_fs = dst_p.reshape(NS, k2, EB)

    w2p = jnp.zeros((hid, cp), jnp.float32).at[:, :ncls].set(W2)
    b1r = b1.reshape(1, hid)
    b2r = b2.reshape(1, ncls)

    stripe = n_pad // NS
    zeros1 = jnp.zeros((stripe, 1), jnp.float32)
    ones_eb = jnp.ones((EB, 1), jnp.float32)
    zeros_fh = jnp.zeros((stripe, fh), jnp.float32)
    zeros_c = jnp.zeros((stripe, cp), jnp.float32)

    degp = _make_degree(n_pad, k)(dst3, ones_eb, zeros1)

    w1s = jnp.stack([W1[:, :fh], W1[:, fh:]])
    dis, table1 = _run_prologue(degp, x, w1s, blk, n_pad)
    table1f = table1.reshape(NC * n_pad, fh)
    s1 = _make_scatter_fs(n_pad, hid, k2)(table1f, src_fs, dst_fs, zeros_fh)
    h2p = _run_mid(s1, table1, dis, b1r, w2p, blk, n)
    s2 = _make_scatter(n_pad, cp, k)(h2p, src3, dst3, zeros_c)
    return _run_epilogue(s2, h2p, dis, b2r, blk, n)


# speedup vs baseline: 1.1025x; 1.1025x over previous
import functools

import jax
import jax.numpy as jnp
from jax import lax
from jax.experimental import pallas as pl
from jax.experimental.pallas import tpu as pltpu
from jax.experimental.pallas import tpu_sc as plsc

NC = 2
NS = 16
NW = NC * NS
EB = 128
RB = 4


def _sc_mesh():
    return plsc.VectorSubcoreMesh(
        core_axis_name="c", subcore_axis_name="s", num_cores=NC, num_subcores=NS
    )


def _degree_body(dst_hbm, ones_hbm, zeros_hbm, out_hbm, idx_v, ones_v, acc_sh, sem):
    c = lax.axis_index("c")
    s = lax.axis_index("s")
    wid = c * NS + s
    k = idx_v.shape[0]
    stripe = acc_sh.shape[0] // NS

    pltpu.sync_copy(dst_hbm.at[wid], idx_v)
    pltpu.sync_copy(ones_hbm, ones_v)
    pltpu.sync_copy(zeros_hbm, acc_sh.at[pl.ds(s * stripe, stripe)])
    plsc.subcore_barrier()

    def fire(j, carry):
        pltpu.async_copy(ones_v, acc_sh.at[idx_v.at[j]], sem, add=True)
        return carry

    lax.fori_loop(0, k, fire, 0)

    def drain(j, carry):
        pltpu.make_async_copy(ones_v, acc_sh.at[idx_v.at[j]], sem).wait()
        return carry

    lax.fori_loop(0, k, drain, 0)
    plsc.subcore_barrier()
    pltpu.sync_copy(
        acc_sh.at[pl.ds(s * stripe, stripe)],
        out_hbm.at[c, pl.ds(s * stripe, stripe)],
    )


def _make_degree(n_pad, k):
    return pl.kernel(
        _degree_body,
        out_type=jax.ShapeDtypeStruct((NC, n_pad), jnp.float32),
        mesh=_sc_mesh(),
        scratch_types=[
            pltpu.VMEM((k, EB), jnp.int32),
            pltpu.VMEM((EB,), jnp.float32),
            pltpu.VMEM_SHARED((n_pad,), jnp.float32),
            pltpu.SemaphoreType.DMA,
        ],
    )


def _scatter_body(table_hbm, src_hbm, dst_hbm, zeros_hbm, out_hbm,
                  sidx_v, didx_v, rows_a, rows_b, rows_c, rows_d, acc_sh,
                  gsem_a, gsem_b, gsem_c, gsem_d,
                  ssem_a, ssem_b, ssem_c, ssem_d):
    c = lax.axis_index("c")
    s = lax.axis_index("s")
    wid = c * NS + s
    k = sidx_v.shape[0]
    stripe = acc_sh.shape[0] // NS
    rows = [rows_a, rows_b, rows_c, rows_d]
    gsems = [gsem_a, gsem_b, gsem_c, gsem_d]
    ssems = [ssem_a, ssem_b, ssem_c, ssem_d]

    pltpu.sync_copy(src_hbm.at[wid], sidx_v)
    pltpu.sync_copy(dst_hbm.at[wid], didx_v)
    for r in range(RB):
        pltpu.async_copy(table_hbm.at[sidx_v.at[r]], rows[r], gsems[r])
    pltpu.sync_copy(zeros_hbm, acc_sh.at[pl.ds(s * stripe, stripe)])
    plsc.subcore_barrier()

    def body(g, carry):
        j0 = g * RB
        for r in range(RB):
            pltpu.make_async_copy(
                table_hbm.at[sidx_v.at[j0 + r]], rows[r], gsems[r]).wait()
            pltpu.async_copy(rows[r], acc_sh.at[didx_v.at[j0 + r]], ssems[r],
                             add=True)
        for r in range(RB):
            pltpu.make_async_copy(
                rows[r], acc_sh.at[didx_v.at[j0 + r]], ssems[r]).wait()

            @pl.when(j0 + RB + r < k)
            def _():
                pltpu.async_copy(
                    table_hbm.at[sidx_v.at[j0 + RB + r]], rows[r], gsems[r])

        return carry

    lax.fori_loop(0, k // RB, body, 0)
    plsc.subcore_barrier()
    pltpu.sync_copy(
        acc_sh.at[pl.ds(s * stripe, stripe)],
        out_hbm.at[c, pl.ds(s * stripe, stripe)],
    )


def _make_scatter(n_pad, feat, k):
    return pl.kernel(
        _scatter_body,
        out_type=jax.ShapeDtypeStruct((NC, n_pad, feat), jnp.float32),
        mesh=_sc_mesh(),
        scratch_types=(
            [
                pltpu.VMEM((k, EB), jnp.int32),
                pltpu.VMEM((k, EB), jnp.int32),
            ]
            + [pltpu.VMEM((EB, feat), jnp.float32)] * RB
            + [pltpu.VMEM_SHARED((n_pad, feat), jnp.float32)]
            + [pltpu.SemaphoreType.DMA] * (2 * RB)
        ),
        compiler_params=pltpu.CompilerParams(use_tc_tiling_on_sc=False),
    )


def _scatter_fs_body(table_hbm, src_hbm, dst_hbm, zeros_hbm, out_hbm,
                     sidx_v, didx_v, rows_a, rows_b, rows_c, rows_d, acc_sh,
                     gsem_a, gsem_b, gsem_c, gsem_d,
                     ssem_a, ssem_b, ssem_c, ssem_d):
    c = lax.axis_index("c")
    s = lax.axis_index("s")
    k = sidx_v.shape[0]
    n_pad, fh = acc_sh.shape
    stripe = n_pad // NS
    rows = [rows_a, rows_b, rows_c, rows_d]
    gsems = [gsem_a, gsem_b, gsem_c, gsem_d]
    ssems = [ssem_a, ssem_b, ssem_c, ssem_d]

    pltpu.sync_copy(src_hbm.at[c, s], sidx_v)
    pltpu.sync_copy(dst_hbm.at[s], didx_v)
    for r in range(RB):
        pltpu.async_copy(table_hbm.at[sidx_v.at[r]], rows[r], gsems[r])
    pltpu.sync_copy(zeros_hbm, acc_sh.at[pl.ds(s * stripe, stripe)])
    plsc.subcore_barrier()

    def body(g, carry):
        j0 = g * RB
        for r in range(RB):
            pltpu.make_async_copy(
                table_hbm.at[sidx_v.at[j0 + r]], rows[r], gsems[r]).wait()
            pltpu.async_copy(rows[r], acc_sh.at[didx_v.at[j0 + r]], ssems[r],
                             add=True)
        for r in range(RB):
            pltpu.make_async_copy(
                rows[r], acc_sh.at[didx_v.at[j0 + r]], ssems[r]).wait()

            @pl.when(j0 + RB + r < k)
            def _():
                pltpu.async_copy(
                    table_hbm.at[sidx_v.at[j0 + RB + r]], rows[r], gsems[r])

        return carry

    lax.fori_loop(0, k // RB, body, 0)
    plsc.subcore_barrier()
    pltpu.sync_copy(
        acc_sh.at[pl.ds(s * stripe, stripe)],
        out_hbm.at[pl.ds(s * stripe, stripe), pl.ds(c * fh, fh)],
    )


def _make_scatter_fs(n_pad, feat, k):
    fh = feat // NC
    return pl.kernel(
        _scatter_fs_body,
        out_type=jax.ShapeDtypeStruct((n_pad, feat), jnp.float32),
        mesh=_sc_mesh(),
        scratch_types=(
            [
                pltpu.VMEM((k, EB), jnp.int32),
                pltpu.VMEM((k, EB), jnp.int32),
            ]
            + [pltpu.VMEM((EB, fh), jnp.float32)] * RB
            + [pltpu.VMEM_SHARED((n_pad, fh), jnp.float32)]
            + [pltpu.SemaphoreType.DMA] * (2 * RB)
        ),
        compiler_params=pltpu.CompilerParams(use_tc_tiling_on_sc=False),
    )


def _prologue_body(deg_ref, x_ref, w_ref, dis_ref, t_ref, *, fh):
    deg = deg_ref[0] + deg_ref[1] + 1.0
    dis = lax.rsqrt(deg)[:, None]
    dis_ref[...] = dis
    hh = jnp.dot(x_ref[...], w_ref[...], preferred_element_type=jnp.float32) * dis
    t_ref[...] = jnp.stack([hh[:, :fh], hh[:, fh:]])


def _mid_body(s_ref, t_ref, dis_ref, b_ref, w_ref, o_ref, *, blk, n_valid):
    i = pl.program_id(0)
    dis = dis_ref[...]
    h = jnp.concatenate([t_ref[0], t_ref[1]], axis=1)
    z = dis * (s_ref[...] + h) + b_ref[...]
    z = jnp.maximum(z, 0.0)
    rows = i * blk + lax.broadcasted_iota(jnp.int32, (blk, 1), 0)
    z = jnp.where(rows < n_valid, z, 0.0)
    o_ref[...] = jnp.dot(
        z, w_ref[...], preferred_element_type=jnp.float32
    ) * dis


def _epilogue_body(s_ref, h_ref, dis_ref, b_ref, o_ref, *, ncls):
    full = dis_ref[...] * (s_ref[0] + s_ref[1] + h_ref[...])
    o_ref[...] = full[:, :ncls] + b_ref[...]


def _row_spec(blk, width):
    return pl.BlockSpec((blk, width), lambda i: (i, 0))


def _const_spec(shape):
    return pl.BlockSpec(shape, lambda i: (0, 0))


def _run_prologue(degp, x, w, blk, n_pad):
    fin = x.shape[1]
    h = w.shape[1]
    fh = h // NC
    grid = n_pad // blk
    return pl.pallas_call(
        functools.partial(_prologue_body, fh=fh),
        grid=(grid,),
        in_specs=[
            pl.BlockSpec((NC, blk), lambda i: (0, i)),
            pl.BlockSpec((blk, fin), lambda i: (i, 0)),
            _const_spec((fin, h)),
        ],
        out_specs=[
            pl.BlockSpec((blk, 1), lambda i: (i, 0)),
            pl.BlockSpec((NC, blk, fh), lambda i: (0, i, 0)),
        ],
        out_shape=[
            jax.ShapeDtypeStruct((n_pad, 1), jnp.float32),
            jax.ShapeDtypeStruct((NC, n_pad, fh), jnp.float32),
        ],
    )(degp, x, w)


def _run_mid(s1, table1, dis, b1, w2p, blk, n_valid):
    n_pad, h = s1.shape
    fh = table1.shape[2]
    cp = w2p.shape[1]
    grid = n_pad // blk
    return pl.pallas_call(
        functools.partial(_mid_body, blk=blk, n_valid=n_valid),
        grid=(grid,),
        in_specs=[
            _row_spec(blk, h),
            pl.BlockSpec((NC, blk, fh), lambda i: (0, i, 0)),
            _row_spec(blk, 1),
            _const_spec((1, h)),
            _const_spec((h, cp)),
        ],
        out_specs=_row_spec(blk, cp),
        out_shape=jax.ShapeDtypeStruct((n_pad, cp), jnp.float32),
    )(s1, table1, dis, b1, w2p)


def _run_epilogue(s_parts, h2p, dis, b2, blk, n):
    n_pad, cp = h2p.shape
    ncls = b2.shape[1]
    grid = n_pad // blk
    return pl.pallas_call(
        functools.partial(_epilogue_body, ncls=ncls),
        grid=(grid,),
        in_specs=[
            pl.BlockSpec((NC, blk, cp), lambda i: (0, i, 0)),
            _row_spec(blk, cp),
            _row_spec(blk, 1),
            _const_spec((1, ncls)),
        ],
        out_specs=pl.BlockSpec((blk, ncls), lambda i: (i, 0)),
        out_shape=jax.ShapeDtypeStruct((n, ncls), jnp.float32),
    )(s_parts, h2p, dis, b2)


def _round_up(a, b):
    return (a + b - 1) // b * b


def kernel(x, edge_index, W1, b1, W2, b2):
    n, fin = x.shape
    hid = W1.shape[1]
    ncls = W2.shape[1]
    e = edge_index.shape[1]

    blk = 1024
    n_pad = _round_up(n + 1, max(blk, NS * 8))
    e_pad = _round_up(e, NW * EB * RB)
    k = e_pad // (NW * EB)
    k2 = e_pad // (NS * EB)
    cp = _round_up(ncls, 16)
    fh = hid // NC

    src = edge_index[0].astype(jnp.int32)
    dst = edge_index[1].astype(jnp.int32)
    pad_idx = n + jnp.arange(e_pad - e, dtype=jnp.int32) % (n_pad - n)
    src_p = jnp.concatenate([src, pad_idx])
    dst_p = jnp.concatenate([dst, pad_idx])
    src3 = src_p.reshape(NW, k, EB)
    dst3 = dst_p.reshape(NW, k, EB)
    src_fs = jnp.stack([src_p, src_p + n_pad]).reshape(NC, NS, k2, EB)
    dst_fs = dst_p.reshape(NS, k2, EB)

    w2p = jnp.zeros((hid, cp), jnp.float32).at[:, :ncls].set(W2)
    b1r = b1.reshape(1, hid)
    b2r = b2.reshape(1, ncls)

    stripe = n_pad // NS
    zeros1 = jnp.zeros((stripe,), jnp.float32)
    ones_eb = jnp.ones((EB,), jnp.float32)
    zeros_fh = jnp.zeros((stripe, fh), jnp.float32)
    zeros_c = jnp.zeros((stripe, cp), jnp.float32)

    degp = _make_degree(n_pad, k)(dst3, ones_eb, zeros1)

    dis, table1 = _run_prologue(degp, x, W1, blk, n_pad)
    table1f = table1.reshape(NC * n_pad, fh)
    s1 = _make_scatter_fs(n_pad, hid, k2)(table1f, src_fs, dst_fs, zeros_fh)
    h2p = _run_mid(s1, table1, dis, b1r, w2p, blk, n)
    s2 = _make_scatter(n_pad, cp, k)(h2p, src3, dst3, zeros_c)
    return _run_epilogue(s2, h2p, dis, b2r, blk, n)


# speedup vs baseline: 1.1076x; 1.0046x over previous
import functools

import jax
import jax.numpy as jnp
from jax import lax
from jax.experimental import pallas as pl
from jax.experimental.pallas import tpu as pltpu
from jax.experimental.pallas import tpu_sc as plsc

NC = 2
NS = 16
NW = NC * NS
EB = 128
RB = 4


def _sc_mesh():
    return plsc.VectorSubcoreMesh(
        core_axis_name="c", subcore_axis_name="s", num_cores=NC, num_subcores=NS
    )


def _degree_body(dst_hbm, ones_hbm, zeros_hbm, out_hbm, idx_v, ones_v, acc_sh, sem):
    c = lax.axis_index("c")
    s = lax.axis_index("s")
    wid = c * NS + s
    k = idx_v.shape[0]
    stripe = acc_sh.shape[0] // NS

    pltpu.sync_copy(dst_hbm.at[wid], idx_v)
    pltpu.sync_copy(ones_hbm, ones_v)
    pltpu.sync_copy(zeros_hbm, acc_sh.at[pl.ds(s * stripe, stripe)])
    plsc.subcore_barrier()

    def fire(j, carry):
        pltpu.async_copy(ones_v, acc_sh.at[idx_v.at[j]], sem, add=True)
        return carry

    lax.fori_loop(0, k, fire, 0)

    def drain(j, carry):
        pltpu.make_async_copy(ones_v, acc_sh.at[idx_v.at[j]], sem).wait()
        return carry

    lax.fori_loop(0, k, drain, 0)
    plsc.subcore_barrier()
    pltpu.sync_copy(
        acc_sh.at[pl.ds(s * stripe, stripe)],
        out_hbm.at[c, pl.ds(s * stripe, stripe)],
    )


def _make_degree(n_pad, k):
    return pl.kernel(
        _degree_body,
        out_type=jax.ShapeDtypeStruct((NC, n_pad), jnp.float32),
        mesh=_sc_mesh(),
        scratch_types=[
            pltpu.VMEM((k, EB), jnp.int32),
            pltpu.VMEM((EB,), jnp.float32),
            pltpu.VMEM_SHARED((n_pad,), jnp.float32),
            pltpu.SemaphoreType.DMA,
        ],
    )


def _scatter_body(table_hbm, src_hbm, dst_hbm, zeros_hbm, out_hbm,
                  sidx_v, didx_v, rows_a, rows_b, rows_c, rows_d, acc_sh,
                  gsem_a, gsem_b, gsem_c, gsem_d,
                  ssem_a, ssem_b, ssem_c, ssem_d):
    c = lax.axis_index("c")
    s = lax.axis_index("s")
    wid = c * NS + s
    k = sidx_v.shape[0]
    stripe = acc_sh.shape[0] // NS
    rows = [rows_a, rows_b, rows_c, rows_d]
    gsems = [gsem_a, gsem_b, gsem_c, gsem_d]
    ssems = [ssem_a, ssem_b, ssem_c, ssem_d]

    pltpu.sync_copy(src_hbm.at[wid], sidx_v)
    pltpu.sync_copy(dst_hbm.at[wid], didx_v)
    for r in range(RB):
        pltpu.async_copy(table_hbm.at[sidx_v.at[r]], rows[r], gsems[r])
    pltpu.sync_copy(zeros_hbm, acc_sh.at[pl.ds(s * stripe, stripe)])
    plsc.subcore_barrier()

    def body(g, carry):
        j0 = g * RB
        for r in range(RB):
            pltpu.make_async_copy(
                table_hbm.at[sidx_v.at[j0 + r]], rows[r], gsems[r]).wait()
            pltpu.async_copy(rows[r], acc_sh.at[didx_v.at[j0 + r]], ssems[r],
                             add=True)
        for r in range(RB):
            pltpu.make_async_copy(
                rows[r], acc_sh.at[didx_v.at[j0 + r]], ssems[r]).wait()

            @pl.when(j0 + RB + r < k)
            def _():
                pltpu.async_copy(
                    table_hbm.at[sidx_v.at[j0 + RB + r]], rows[r], gsems[r])

        return carry

    lax.fori_loop(0, k // RB, body, 0)
    plsc.subcore_barrier()
    pltpu.sync_copy(
        acc_sh.at[pl.ds(s * stripe, stripe)],
        out_hbm.at[c, pl.ds(s * stripe, stripe)],
    )


def _make_scatter(n_pad, feat, k):
    return pl.kernel(
        _scatter_body,
        out_type=jax.ShapeDtypeStruct((NC, n_pad, feat), jnp.float32),
        mesh=_sc_mesh(),
        scratch_types=(
            [
                pltpu.VMEM((k, EB), jnp.int32),
                pltpu.VMEM((k, EB), jnp.int32),
            ]
            + [pltpu.VMEM((EB, feat), jnp.float32)] * RB
            + [pltpu.VMEM_SHARED((n_pad, feat), jnp.float32)]
            + [pltpu.SemaphoreType.DMA] * (2 * RB)
        ),
        compiler_params=pltpu.CompilerParams(use_tc_tiling_on_sc=False),
    )


def _scatter_fs_body(table_hbm, src_hbm, dst_hbm, zeros_hbm, out_hbm,
                     sidx_v, didx_v, rows_a, rows_b, rows_c, rows_d, acc_sh,
                     gsem_a, gsem_b, gsem_c, gsem_d,
                     ssem_a, ssem_b, ssem_c, ssem_d):
    c = lax.axis_index("c")
    s = lax.axis_index("s")
    k = sidx_v.shape[0]
    n_pad, fh = acc_sh.shape
    stripe = n_pad // NS
    rows = [rows_a, rows_b, rows_c, rows_d]
    gsems = [gsem_a, gsem_b, gsem_c, gsem_d]
    ssems = [ssem_a, ssem_b, ssem_c, ssem_d]

    tview = table_hbm.at[pl.ds(c * n_pad, n_pad)]
    pltpu.sync_copy(src_hbm.at[s], sidx_v)
    pltpu.sync_copy(dst_hbm.at[s], didx_v)
    for r in range(RB):
        pltpu.async_copy(tview.at[sidx_v.at[r]], rows[r], gsems[r])
    pltpu.sync_copy(zeros_hbm, acc_sh.at[pl.ds(s * stripe, stripe)])
    plsc.subcore_barrier()

    def body(g, carry):
        j0 = g * RB
        for r in range(RB):
            pltpu.make_async_copy(
                tview.at[sidx_v.at[j0 + r]], rows[r], gsems[r]).wait()
            pltpu.async_copy(rows[r], acc_sh.at[didx_v.at[j0 + r]], ssems[r],
                             add=True)
        for r in range(RB):
            pltpu.make_async_copy(
                rows[r], acc_sh.at[didx_v.at[j0 + r]], ssems[r]).wait()

            @pl.when(j0 + RB + r < k)
            def _():
                pltpu.async_copy(
                    tview.at[sidx_v.at[j0 + RB + r]], rows[r], gsems[r])

        return carry

    lax.fori_loop(0, k // RB, body, 0)
    plsc.subcore_barrier()
    pltpu.sync_copy(
        acc_sh.at[pl.ds(s * stripe, stripe)],
        out_hbm.at[pl.ds(s * stripe, stripe), pl.ds(c * fh, fh)],
    )


def _make_scatter_fs(n_pad, feat, k):
    fh = feat // NC
    return pl.kernel(
        _scatter_fs_body,
        out_type=jax.ShapeDtypeStruct((n_pad, feat), jnp.float32),
        mesh=_sc_mesh(),
        scratch_types=(
            [
                pltpu.VMEM((k, EB), jnp.int32),
                pltpu.VMEM((k, EB), jnp.int32),
            ]
            + [pltpu.VMEM((EB, fh), jnp.float32)] * RB
            + [pltpu.VMEM_SHARED((n_pad, fh), jnp.float32)]
            + [pltpu.SemaphoreType.DMA] * (2 * RB)
        ),
        compiler_params=pltpu.CompilerParams(use_tc_tiling_on_sc=False),
    )


def _prologue_body(deg_ref, x_ref, w_ref, dis_ref, t_ref, *, fh):
    deg = deg_ref[0] + deg_ref[1] + 1.0
    dis = lax.rsqrt(deg)[:, None]
    dis_ref[...] = dis
    hh = jnp.dot(x_ref[...], w_ref[...], preferred_element_type=jnp.float32) * dis
    t_ref[...] = jnp.stack([hh[:, :fh], hh[:, fh:]])


def _mid_body(s_ref, t_ref, dis_ref, b_ref, w_ref, o_ref, *, blk, n_valid):
    i = pl.program_id(0)
    dis = dis_ref[...]
    h = jnp.concatenate([t_ref[0], t_ref[1]], axis=1)
    z = dis * (s_ref[...] + h) + b_ref[...]
    z = jnp.maximum(z, 0.0)
    rows = i * blk + lax.broadcasted_iota(jnp.int32, (blk, 1), 0)
    z = jnp.where(rows < n_valid, z, 0.0)
    o_ref[...] = jnp.dot(
        z, w_ref[...], preferred_element_type=jnp.float32
    ) * dis


def _epilogue_body(s_ref, h_ref, dis_ref, b_ref, o_ref, *, ncls):
    full = dis_ref[...] * (s_ref[0] + s_ref[1] + h_ref[...])
    o_ref[...] = full[:, :ncls] + b_ref[...]


def _row_spec(blk, width):
    return pl.BlockSpec((blk, width), lambda i: (i, 0))


def _const_spec(shape):
    return pl.BlockSpec(shape, lambda i: (0, 0))


def _run_prologue(degp, x, w, blk, n_pad):
    fin = x.shape[1]
    h = w.shape[1]
    fh = h // NC
    grid = n_pad // blk
    return pl.pallas_call(
        functools.partial(_prologue_body, fh=fh),
        grid=(grid,),
        in_specs=[
            pl.BlockSpec((NC, blk), lambda i: (0, i)),
            pl.BlockSpec((blk, fin), lambda i: (i, 0)),
            _const_spec((fin, h)),
        ],
        out_specs=[
            pl.BlockSpec((blk, 1), lambda i: (i, 0)),
            pl.BlockSpec((NC, blk, fh), lambda i: (0, i, 0)),
        ],
        out_shape=[
            jax.ShapeDtypeStruct((n_pad, 1), jnp.float32),
            jax.ShapeDtypeStruct((NC, n_pad, fh), jnp.float32),
        ],
    )(degp, x, w)


def _run_mid(s1, table1, dis, b1, w2p, blk, n_valid):
    n_pad, h = s1.shape
    fh = table1.shape[2]
    cp = w2p.shape[1]
    grid = n_pad // blk
    return pl.pallas_call(
        functools.partial(_mid_body, blk=blk, n_valid=n_valid),
        grid=(grid,),
        in_specs=[
            _row_spec(blk, h),
            pl.BlockSpec((NC, blk, fh), lambda i: (0, i, 0)),
            _row_spec(blk, 1),
            _const_spec((1, h)),
            _const_spec((h, cp)),
        ],
        out_specs=_row_spec(blk, cp),
        out_shape=jax.ShapeDtypeStruct((n_pad, cp), jnp.float32),
    )(s1, table1, dis, b1, w2p)


def _run_epilogue(s_parts, h2p, dis, b2, blk, n):
    n_pad, cp = h2p.shape
    ncls = b2.shape[1]
    grid = n_pad // blk
    return pl.pallas_call(
        functools.partial(_epilogue_body, ncls=ncls),
        grid=(grid,),
        in_specs=[
            pl.BlockSpec((NC, blk, cp), lambda i: (0, i, 0)),
            _row_spec(blk, cp),
            _row_spec(blk, 1),
            _const_spec((1, ncls)),
        ],
        out_specs=pl.BlockSpec((blk, ncls), lambda i: (i, 0)),
        out_shape=jax.ShapeDtypeStruct((n, ncls), jnp.float32),
    )(s_parts, h2p, dis, b2)


def _round_up(a, b):
    return (a + b - 1) // b * b


def kernel(x, edge_index, W1, b1, W2, b2):
    n, fin = x.shape
    hid = W1.shape[1]
    ncls = W2.shape[1]
    e = edge_index.shape[1]

    blk = 1024
    n_pad = _round_up(n + 1, max(blk, NS * 8))
    e_pad = _round_up(e, NW * EB * RB)
    k = e_pad // (NW * EB)
    k2 = e_pad // (NS * EB)
    cp = _round_up(ncls, 16)
    fh = hid // NC

    src = edge_index[0].astype(jnp.int32)
    dst = edge_index[1].astype(jnp.int32)
    pad_idx = n + jnp.arange(e_pad - e, dtype=jnp.int32) % (n_pad - n)
    src_p = jnp.concatenate([src, pad_idx])
    dst_p = jnp.concatenate([dst, pad_idx])
    src3 = src_p.reshape(NW, k, EB)
    dst3 = dst_p.reshape(NW, k, EB)
    src_fs = src_p.reshape(NS, k2, EB)
    dst_fs = dst_p.reshape(NS, k2, EB)

    w2p = jnp.zeros((hid, cp), jnp.float32).at[:, :ncls].set(W2)
    b1r = b1.reshape(1, hid)
    b2r = b2.reshape(1, ncls)

    stripe = n_pad // NS
    zeros1 = jnp.zeros((stripe,), jnp.float32)
    ones_eb = jnp.ones((EB,), jnp.float32)
    zeros_fh = jnp.zeros((stripe, fh), jnp.float32)
    zeros_c = jnp.zeros((stripe, cp), jnp.float32)

    degp = _make_degree(n_pad, k)(dst3, ones_eb, zeros1)

    dis, table1 = _run_prologue(degp, x, W1, blk, n_pad)
    table1f = table1.reshape(NC * n_pad, fh)
    s1 = _make_scatter_fs(n_pad, hid, k2)(table1f, src_fs, dst_fs, zeros_fh)
    h2p = _run_mid(s1, table1, dis, b1r, w2p, blk, n)
    s2 = _make_scatter(n_pad, cp, k)(h2p, src3, dst3, zeros_c)
    return _run_epilogue(s2, h2p, dis, b2r, blk, n)
